# baseline (speedup 1.0000x reference)
import functools

import jax
import jax.numpy as jnp
from jax import lax
from jax.experimental import pallas as pl
from jax.experimental.pallas import tpu as pltpu
from jax.experimental.pallas import tpu_sc as plsc

_COLS = 25600
_SC_BLOCKS = 200
_NUM_WORKERS = 32


def _stats_tail_kernel(w_ref, k_ref, wt_ref, xt_ref, stats_ref, tail_ref):
    wv = w_ref[...]
    m = jnp.max(wv)
    d = jnp.sum(jnp.exp(wv - m))
    coeff = jnp.round(k_ref[0, 0]) / (jnp.float32(wv.size) * d)
    stats_ref[...] = jnp.stack([jnp.full((16,), m), jnp.full((16,), coeff)])
    e_t = jnp.exp(wt_ref[...] - m) * coeff
    tail_ref[...] = jax.lax.dot_general(
        e_t, xt_ref[...], (((1,), (0,)), ((), ())),
        preferred_element_type=jnp.float32)


def _stats_kernel(w_ref, k_ref, out_ref):
    wv = w_ref[...]
    m = jnp.max(wv)
    d = jnp.sum(jnp.exp(wv - m))
    coeff = jnp.round(k_ref[0, 0]) / (jnp.float32(wv.size) * d)
    out_ref[...] = jnp.stack([jnp.full((16,), m), jnp.full((16,), coeff)])


def _wsum_tc_grid_kernel(stats_ref, w_ref, x_ref, out_ref):
    i = pl.program_id(0)
    m = stats_ref[0, 0]
    coeff = stats_ref[1, 0]
    e = jnp.exp(w_ref[0] - m) * coeff
    part = jax.lax.dot_general(
        e, x_ref[...], (((1,), (0,)), ((), ())),
        preferred_element_type=jnp.float32)

    @pl.when(i == 0)
    def _init():
        out_ref[...] = jnp.zeros_like(out_ref)

    out_ref[...] += part


def _combine_kernel(tc_ref, sc_ref, out_ref):
    a = sc_ref[...].reshape(_NUM_WORKERS, 64, 16)
    out_ref[...] = tc_ref[...] + jnp.sum(a, axis=(0, 2)).reshape(1, 64)


def _make_sc_wsum(sc0, t):
    mesh = plsc.VectorSubcoreMesh(core_axis_name="c", subcore_axis_name="s")
    max_blk = (_SC_BLOCKS + _NUM_WORKERS - 1) // _NUM_WORKERS

    @functools.partial(
        pl.kernel,
        out_type=jax.ShapeDtypeStruct((_NUM_WORKERS * t * 16,), jnp.float32),
        mesh=mesh,
        scratch_types=[
            pltpu.VMEM((t, 512), jnp.float32),
            pltpu.VMEM((t, 512), jnp.float32),
            pltpu.VMEM((max_blk * 1024,), jnp.float32),
            pltpu.VMEM((2, 16), jnp.float32),
            pltpu.VMEM((t * 16,), jnp.float32),
            pltpu.SemaphoreType.DMA,
            pltpu.SemaphoreType.DMA,
            pltpu.SemaphoreType.DMA,
        ],
    )
    def sc_fn(stats_hbm, w_hbm, xt_hbm, out_hbm,
              xa, xb, eb, sb, acc, sxa, sxb, sw):
        cid = lax.axis_index("c")
        sid = lax.axis_index("s")
        wid = sid * 2 + cid
        nblk = (_SC_BLOCKS - wid + _NUM_WORKERS - 1) // _NUM_WORKERS

        pltpu.sync_copy(stats_hbm, sb)
        mv = sb[0, pl.ds(0, 16)]
        cv = sb[1, pl.ds(0, 16)]

        def load_w(i, carry):
            c0 = sc0 + (wid + i * _NUM_WORKERS) * 1024
            pltpu.async_copy(
                w_hbm.at[pl.ds(c0, 1024)], eb.at[pl.ds(i * 1024, 1024)], sw)
            pltpu.make_async_copy(
                w_hbm.at[pl.ds(0, 1024)], eb.at[pl.ds(0, 1024)], sw).wait()
            return carry
        lax.fori_loop(0, nblk, load_w, 0)

        def expgrp(g, carry):
            ev = eb[pl.ds(g * 16, 16)]
            eb[pl.ds(g * 16, 16)] = jnp.exp(ev - mv) * cv
            return carry
        lax.fori_loop(0, nblk * 64, expgrp, 0)

        def zgrp(i, carry):
            acc[pl.ds(i * 16, 16)] = jnp.zeros((16,), jnp.float32)
            return carry
        lax.fori_loop(0, t, zgrp, 0)

        xbufs = (xa, xb)
        sxs = (sxa, sxb)

        def col0_of(q):
            blk = wid + (q // 2) * _NUM_WORKERS
            return sc0 + blk * 1024 + (q % 2) * 512

        def start(q, p):
            pltpu.async_copy(
                xt_hbm.at[:, pl.ds(col0_of(q), 512)], xbufs[p], sxs[p])

        def wait(p):
            pltpu.make_async_copy(
                xt_hbm.at[:, pl.ds(0, 512)], xbufs[p], sxs[p]).wait()

        def process(q, p):
            ebase = (q // 2) * 1024 + (q % 2) * 512

            def colgrp(g, carry):
                ev = eb[pl.ds(ebase + g * 16, 16)]
                for tt in range(t):
                    prod = ev * xbufs[p][tt, pl.ds(g * 16, 16)]
                    plsc.addupdate(acc.at[pl.ds(tt * 16, 16)], prod)
                return carry
            lax.fori_loop(0, 32, colgrp, 0)

        nq = 2 * nblk
        start(0, 0)
        start(1, 1)

        def body(i, carry):
            q0 = 2 * i
            wait(0)
            process(q0, 0)

            @pl.when(q0 + 2 < nq)
            def _():
                start(q0 + 2, 0)

            wait(1)
            process(q0 + 1, 1)

            @pl.when(q0 + 3 < nq)
            def _():
                start(q0 + 3, 1)

            return carry

        lax.fori_loop(0, nq // 2, body, 0)
        pltpu.sync_copy(acc, out_hbm.at[pl.ds(wid * t * 16, t * 16)])

    return sc_fn


def _make_xt_kernel(n, t, b, nb):
    def body(stats_ref, tailp_ref, w_ref, xt_ref, out_ref,
             xa, xb_, wa, wb, sb, tb, acc, sxa, sxb, swa, swb):
        pltpu.sync_copy(stats_ref, sb)
        pltpu.sync_copy(tailp_ref, tb)
        m = sb[0, 0]
        coeff = sb[1, 0]

        xbufs = (xa, xb_)
        wbufs = (wa, wb)
        sxs = (sxa, sxb)
        sws = (swa, swb)

        def start(j, p):
            pltpu.async_copy(xt_ref.at[:, pl.ds(j * b, b)], xbufs[p], sxs[p])
            pltpu.async_copy(w_ref.at[pl.ds(j * b, b)], wbufs[p], sws[p])

        def wait(p):
            pltpu.make_async_copy(
                xt_ref.at[:, pl.ds(0, b)], xbufs[p], sxs[p]).wait()
            pltpu.make_async_copy(
                w_ref.at[pl.ds(0, b)], wbufs[p], sws[p]).wait()

        def process(p):
            e = (jnp.exp(wbufs[p][...] - m) * coeff).reshape(1, b)
            acc[...] += xbufs[p][...] * e

        start(0, 0)
        if nb > 1:
            start(1, 1)
        acc[...] = jnp.zeros_like(acc)

        def pair(i, carry):
            j0 = 2 * i
            wait(0)
            process(0)

            @pl.when(j0 + 2 < nb)
            def _():
                start(j0 + 2, 0)

            wait(1)
            process(1)

            @pl.when(j0 + 3 < nb)
            def _():
                start(j0 + 3, 1)

            return carry

        jax.lax.fori_loop(0, nb // 2, pair, 0)
        if nb % 2 == 1:
            wait((nb - 1) % 2)
            process((nb - 1) % 2)
        out_ref[...] = jnp.sum(acc[...], axis=1).reshape(1, t) + tb[...]

    return pl.pallas_call(
        body,
        out_shape=jax.ShapeDtypeStruct((1, t), jnp.float32),
        in_specs=[
            pl.BlockSpec(memory_space=pltpu.HBM),
            pl.BlockSpec(memory_space=pltpu.HBM),
            pl.BlockSpec(memory_space=pltpu.HBM),
            pl.BlockSpec(memory_space=pltpu.HBM),
        ],
        out_specs=pl.BlockSpec((1, t), lambda: (0, 0)),
        scratch_shapes=[
            pltpu.VMEM((t, b), jnp.float32),
            pltpu.VMEM((t, b), jnp.float32),
            pltpu.VMEM((b,), jnp.float32),
            pltpu.VMEM((b,), jnp.float32),
            pltpu.VMEM((2, 16), jnp.float32),
            pltpu.VMEM((1, t), jnp.float32),
            pltpu.VMEM((t, b), jnp.float32),
            pltpu.SemaphoreType.DMA,
            pltpu.SemaphoreType.DMA,
            pltpu.SemaphoreType.DMA,
            pltpu.SemaphoreType.DMA,
        ],
    )


def _pick_block(n):
    for b in (8000, 10000, 5000, 4096, 4000, 2048, 2000, 1000):
        if n % b == 0:
            return b
    return n


def kernel(x, w, k_param):
    n, t = x.shape
    rows = 1000 if n % 1000 == 0 else 1
    w2d = w.reshape(n // rows, rows)
    k2d = k_param.reshape(1, 1)

    bc = _COLS
    nb = n // bc
    tail = n - nb * bc
    sc_cols = _SC_BLOCKS * 1024
    use_xt = (t == 64 and nb >= 2 and tail % 8 == 0 and tail > 0
              and bc % 1024 == 0 and sc_cols % bc == 0
              and nb * bc - sc_cols >= 2 * bc)

    if use_xt:
        wt = w[n - tail:].reshape(1, tail)
        xtail = x[n - tail:]
        stats, tailp = pl.pallas_call(
            _stats_tail_kernel,
            out_shape=(
                jax.ShapeDtypeStruct((2, 16), jnp.float32),
                jax.ShapeDtypeStruct((1, t), jnp.float32),
            ),
            in_specs=[
                pl.BlockSpec((n // rows, rows), lambda: (0, 0)),
                pl.BlockSpec((1, 1), lambda: (0, 0)),
                pl.BlockSpec((1, tail), lambda: (0, 0)),
                pl.BlockSpec((tail, t), lambda: (0, 0)),
            ],
            out_specs=(
                pl.BlockSpec((2, 16), lambda: (0, 0)),
                pl.BlockSpec((1, t), lambda: (0, 0)),
            ),
        )(w2d, k2d, wt, xtail)
        sc0 = nb * bc - sc_cols
        nb_tc = sc0 // bc
        sc_part = _make_sc_wsum(sc0, t)(stats, w, x.T)
        tc_part = _make_xt_kernel(n, t, bc, nb_tc)(stats, tailp, w, x.T)
        out = pl.pallas_call(
            _combine_kernel,
            out_shape=jax.ShapeDtypeStruct((1, t), jnp.float32),
            in_specs=[
                pl.BlockSpec((1, t), lambda: (0, 0)),
                pl.BlockSpec((_NUM_WORKERS * t, 16), lambda: (0, 0)),
            ],
            out_specs=pl.BlockSpec((1, t), lambda: (0, 0)),
        )(tc_part, sc_part.reshape(_NUM_WORKERS * t, 16))
    else:
        stats = pl.pallas_call(
            _stats_kernel,
            out_shape=jax.ShapeDtypeStruct((2, 16), jnp.float32),
            in_specs=[
                pl.BlockSpec((n // rows, rows), lambda: (0, 0)),
                pl.BlockSpec((1, 1), lambda: (0, 0)),
            ],
            out_specs=pl.BlockSpec((2, 16), lambda: (0, 0)),
        )(w2d, k2d)
        b = _pick_block(n)
        out = pl.pallas_call(
            _wsum_tc_grid_kernel,
            grid=(n // b,),
            out_shape=jax.ShapeDtypeStruct((1, t), jnp.float32),
            in_specs=[
                pl.BlockSpec((2, 16), lambda i: (0, 0)),
                pl.BlockSpec((1, 1, b), lambda i: (i, 0, 0)),
                pl.BlockSpec((b, t), lambda i: (i, 0)),
            ],
            out_specs=pl.BlockSpec((1, t), lambda i: (0, 0)),
        )(stats, w.reshape(n // b, 1, b), x)

    return out.reshape(t)


# speedup vs baseline: 1.4880x; 1.4880x over previous
import functools

import jax
import jax.numpy as jnp
from jax import lax
from jax.experimental import pallas as pl
from jax.experimental.pallas import tpu as pltpu
from jax.experimental.pallas import tpu_sc as plsc

_COLS = 25600
_SC_BLOCKS = 100
_NUM_WORKERS = 32


def _stats_tail_kernel(w_ref, k_ref, wt_ref, xt_ref, stats_ref, tail_ref):
    wv = w_ref[...]
    m = jnp.max(wv)
    d = jnp.sum(jnp.exp(wv - m))
    coeff = jnp.round(k_ref[0, 0]) / (jnp.float32(wv.size) * d)
    stats_ref[...] = jnp.stack([jnp.full((16,), m), jnp.full((16,), coeff)])
    e_t = jnp.exp(wt_ref[...] - m) * coeff
    tail_ref[...] = jax.lax.dot_general(
        e_t, xt_ref[...], (((1,), (0,)), ((), ())),
        preferred_element_type=jnp.float32)


def _stats_kernel(w_ref, k_ref, out_ref):
    wv = w_ref[...]
    m = jnp.max(wv)
    d = jnp.sum(jnp.exp(wv - m))
    coeff = jnp.round(k_ref[0, 0]) / (jnp.float32(wv.size) * d)
    out_ref[...] = jnp.stack([jnp.full((16,), m), jnp.full((16,), coeff)])


def _wsum_tc_grid_kernel(stats_ref, w_ref, x_ref, out_ref):
    i = pl.program_id(0)
    m = stats_ref[0, 0]
    coeff = stats_ref[1, 0]
    e = jnp.exp(w_ref[0] - m) * coeff
    part = jax.lax.dot_general(
        e, x_ref[...], (((1,), (0,)), ((), ())),
        preferred_element_type=jnp.float32)

    @pl.when(i == 0)
    def _init():
        out_ref[...] = jnp.zeros_like(out_ref)

    out_ref[...] += part


def _combine_kernel(tc_ref, sc_ref, out_ref):
    a = sc_ref[...].reshape(_NUM_WORKERS, 64, 16)
    out_ref[...] = tc_ref[...] + jnp.sum(a, axis=(0, 2)).reshape(1, 64)


def _make_sc_wsum(sc0, t):
    mesh = plsc.VectorSubcoreMesh(core_axis_name="c", subcore_axis_name="s")
    max_blk = (_SC_BLOCKS + _NUM_WORKERS - 1) // _NUM_WORKERS

    @functools.partial(
        pl.kernel,
        out_type=jax.ShapeDtypeStruct((_NUM_WORKERS * t * 16,), jnp.float32),
        mesh=mesh,
        scratch_types=[
            pltpu.VMEM((t, 512), jnp.float32),
            pltpu.VMEM((t, 512), jnp.float32),
            pltpu.VMEM((max_blk * 1024,), jnp.float32),
            pltpu.VMEM((2, 16), jnp.float32),
            pltpu.VMEM((t * 16,), jnp.float32),
            pltpu.SemaphoreType.DMA,
            pltpu.SemaphoreType.DMA,
            pltpu.SemaphoreType.DMA,
        ],
    )
    def sc_fn(stats_hbm, w_hbm, xt_hbm, out_hbm,
              xa, xb, eb, sb, acc, sxa, sxb, sw):
        cid = lax.axis_index("c")
        sid = lax.axis_index("s")
        wid = sid * 2 + cid
        nblk = (_SC_BLOCKS - wid + _NUM_WORKERS - 1) // _NUM_WORKERS

        pltpu.sync_copy(stats_hbm, sb)
        mv = sb[0, pl.ds(0, 16)]
        cv = sb[1, pl.ds(0, 16)]

        def load_w(i, carry):
            c0 = sc0 + (wid + i * _NUM_WORKERS) * 1024
            pltpu.async_copy(
                w_hbm.at[pl.ds(c0, 1024)], eb.at[pl.ds(i * 1024, 1024)], sw)
            pltpu.make_async_copy(
                w_hbm.at[pl.ds(0, 1024)], eb.at[pl.ds(0, 1024)], sw).wait()
            return carry
        lax.fori_loop(0, nblk, load_w, 0)

        def expgrp(g, carry):
            ev = eb[pl.ds(g * 16, 16)]
            eb[pl.ds(g * 16, 16)] = jnp.exp(ev - mv) * cv
            return carry
        lax.fori_loop(0, nblk * 64, expgrp, 0)

        def zgrp(i, carry):
            acc[pl.ds(i * 16, 16)] = jnp.zeros((16,), jnp.float32)
            return carry
        lax.fori_loop(0, t, zgrp, 0)

        xbufs = (xa, xb)
        sxs = (sxa, sxb)

        def col0_of(q):
            blk = wid + (q // 2) * _NUM_WORKERS
            return sc0 + blk * 1024 + (q % 2) * 512

        def start(q, p):
            pltpu.async_copy(
                xt_hbm.at[:, pl.ds(col0_of(q), 512)], xbufs[p], sxs[p])

        def wait(p):
            pltpu.make_async_copy(
                xt_hbm.at[:, pl.ds(0, 512)], xbufs[p], sxs[p]).wait()

        def process(q, p):
            ebase = (q // 2) * 1024 + (q % 2) * 512

            def colgrp(g, carry):
                ev = eb[pl.ds(ebase + g * 16, 16)]
                for tt in range(t):
                    prod = ev * xbufs[p][tt, pl.ds(g * 16, 16)]
                    plsc.addupdate(acc.at[pl.ds(tt * 16, 16)], prod)
                return carry
            lax.fori_loop(0, 32, colgrp, 0)

        nq = 2 * nblk
        start(0, 0)
        start(1, 1)

        def body(i, carry):
            q0 = 2 * i
            wait(0)
            process(q0, 0)

            @pl.when(q0 + 2 < nq)
            def _():
                start(q0 + 2, 0)

            wait(1)
            process(q0 + 1, 1)

            @pl.when(q0 + 3 < nq)
            def _():
                start(q0 + 3, 1)

            return carry

        lax.fori_loop(0, nq // 2, body, 0)
        pltpu.sync_copy(acc, out_hbm.at[pl.ds(wid * t * 16, t * 16)])

    return sc_fn


def _make_xt_kernel(n, t, b, nb):
    def body(stats_ref, tailp_ref, w_ref, xt_ref, out_ref,
             xa, xb_, wa, wb, sb, tb, acc, sxa, sxb, swa, swb):
        pltpu.sync_copy(stats_ref, sb)
        pltpu.sync_copy(tailp_ref, tb)
        m = sb[0, 0]
        coeff = sb[1, 0]

        xbufs = (xa, xb_)
        wbufs = (wa, wb)
        sxs = (sxa, sxb)
        sws = (swa, swb)

        def start(j, p):
            pltpu.async_copy(xt_ref.at[:, pl.ds(j * b, b)], xbufs[p], sxs[p])
            pltpu.async_copy(w_ref.at[pl.ds(j * b, b)], wbufs[p], sws[p])

        def wait(p):
            pltpu.make_async_copy(
                xt_ref.at[:, pl.ds(0, b)], xbufs[p], sxs[p]).wait()
            pltpu.make_async_copy(
                w_ref.at[pl.ds(0, b)], wbufs[p], sws[p]).wait()

        def process(p):
            e = (jnp.exp(wbufs[p][...] - m) * coeff).reshape(1, b)
            acc[...] += xbufs[p][...] * e

        start(0, 0)
        if nb > 1:
            start(1, 1)
        acc[...] = jnp.zeros_like(acc)

        def pair(i, carry):
            j0 = 2 * i
            wait(0)
            process(0)

            @pl.when(j0 + 2 < nb)
            def _():
                start(j0 + 2, 0)

            wait(1)
            process(1)

            @pl.when(j0 + 3 < nb)
            def _():
                start(j0 + 3, 1)

            return carry

        jax.lax.fori_loop(0, nb // 2, pair, 0)
        if nb % 2 == 1:
            wait((nb - 1) % 2)
            process((nb - 1) % 2)
        out_ref[...] = jnp.sum(acc[...], axis=1).reshape(1, t) + tb[...]

    return pl.pallas_call(
        body,
        out_shape=jax.ShapeDtypeStruct((1, t), jnp.float32),
        in_specs=[
            pl.BlockSpec(memory_space=pltpu.HBM),
            pl.BlockSpec(memory_space=pltpu.HBM),
            pl.BlockSpec(memory_space=pltpu.HBM),
            pl.BlockSpec(memory_space=pltpu.HBM),
        ],
        out_specs=pl.BlockSpec((1, t), lambda: (0, 0)),
        scratch_shapes=[
            pltpu.VMEM((t, b), jnp.float32),
            pltpu.VMEM((t, b), jnp.float32),
            pltpu.VMEM((b,), jnp.float32),
            pltpu.VMEM((b,), jnp.float32),
            pltpu.VMEM((2, 16), jnp.float32),
            pltpu.VMEM((1, t), jnp.float32),
            pltpu.VMEM((t, b), jnp.float32),
            pltpu.SemaphoreType.DMA,
            pltpu.SemaphoreType.DMA,
            pltpu.SemaphoreType.DMA,
            pltpu.SemaphoreType.DMA,
        ],
    )


def _pick_block(n):
    for b in (8000, 10000, 5000, 4096, 4000, 2048, 2000, 1000):
        if n % b == 0:
            return b
    return n


def kernel(x, w, k_param):
    n, t = x.shape
    rows = 1000 if n % 1000 == 0 else 1
    w2d = w.reshape(n // rows, rows)
    k2d = k_param.reshape(1, 1)

    bc = _COLS
    nb = n // bc
    tail = n - nb * bc
    sc_cols = _SC_BLOCKS * 1024
    use_xt = (t == 64 and nb >= 2 and tail % 8 == 0 and tail > 0
              and bc % 1024 == 0 and sc_cols % bc == 0
              and nb * bc - sc_cols >= 2 * bc)

    if use_xt:
        wt = w[n - tail:].reshape(1, tail)
        xtail = x[n - tail:]
        stats, tailp = pl.pallas_call(
            _stats_tail_kernel,
            out_shape=(
                jax.ShapeDtypeStruct((2, 16), jnp.float32),
                jax.ShapeDtypeStruct((1, t), jnp.float32),
            ),
            in_specs=[
                pl.BlockSpec((n // rows, rows), lambda: (0, 0)),
                pl.BlockSpec((1, 1), lambda: (0, 0)),
                pl.BlockSpec((1, tail), lambda: (0, 0)),
                pl.BlockSpec((tail, t), lambda: (0, 0)),
            ],
            out_specs=(
                pl.BlockSpec((2, 16), lambda: (0, 0)),
                pl.BlockSpec((1, t), lambda: (0, 0)),
            ),
        )(w2d, k2d, wt, xtail)
        sc0 = nb * bc - sc_cols
        nb_tc = sc0 // bc
        sc_part = _make_sc_wsum(sc0, t)(stats, w, x.T)
        tc_part = _make_xt_kernel(n, t, bc, nb_tc)(stats, tailp, w, x.T)
        out = pl.pallas_call(
            _combine_kernel,
            out_shape=jax.ShapeDtypeStruct((1, t), jnp.float32),
            in_specs=[
                pl.BlockSpec((1, t), lambda: (0, 0)),
                pl.BlockSpec((_NUM_WORKERS * t, 16), lambda: (0, 0)),
            ],
            out_specs=pl.BlockSpec((1, t), lambda: (0, 0)),
        )(tc_part, sc_part.reshape(_NUM_WORKERS * t, 16))
    else:
        stats = pl.pallas_call(
            _stats_kernel,
            out_shape=jax.ShapeDtypeStruct((2, 16), jnp.float32),
            in_specs=[
                pl.BlockSpec((n // rows, rows), lambda: (0, 0)),
                pl.BlockSpec((1, 1), lambda: (0, 0)),
            ],
            out_specs=pl.BlockSpec((2, 16), lambda: (0, 0)),
        )(w2d, k2d)
        b = _pick_block(n)
        out = pl.pallas_call(
            _wsum_tc_grid_kernel,
            grid=(n // b,),
            out_shape=jax.ShapeDtypeStruct((1, t), jnp.float32),
            in_specs=[
                pl.BlockSpec((2, 16), lambda i: (0, 0)),
                pl.BlockSpec((1, 1, b), lambda i: (i, 0, 0)),
                pl.BlockSpec((b, t), lambda i: (i, 0)),
            ],
            out_specs=pl.BlockSpec((1, t), lambda i: (0, 0)),
        )(stats, w.reshape(n // b, 1, b), x)

    return out.reshape(t)


# speedup vs baseline: 1.9160x; 1.2876x over previous
import jax
import jax.numpy as jnp
from jax.experimental import pallas as pl
from jax.experimental.pallas import tpu as pltpu

_COLS = 25600


def _stats_tail_kernel(w_ref, k_ref, wt_ref, xt_ref, stats_ref, tail_ref):
    wv = w_ref[...]
    m = jnp.max(wv)
    d = jnp.sum(jnp.exp(wv - m))
    coeff = jnp.round(k_ref[0, 0]) / (jnp.float32(wv.size) * d)
    stats_ref[...] = jnp.stack([jnp.full((16,), m), jnp.full((16,), coeff)])
    e_t = jnp.exp(wt_ref[...] - m) * coeff
    tail_ref[...] = jax.lax.dot_general(
        e_t, xt_ref[...], (((1,), (0,)), ((), ())),
        preferred_element_type=jnp.float32)


def _stats_kernel(w_ref, k_ref, out_ref):
    wv = w_ref[...]
    m = jnp.max(wv)
    d = jnp.sum(jnp.exp(wv - m))
    coeff = jnp.round(k_ref[0, 0]) / (jnp.float32(wv.size) * d)
    out_ref[...] = jnp.stack([jnp.full((16,), m), jnp.full((16,), coeff)])


def _wsum_tc_grid_kernel(stats_ref, w_ref, x_ref, out_ref):
    i = pl.program_id(0)
    m = stats_ref[0, 0]
    coeff = stats_ref[1, 0]
    e = jnp.exp(w_ref[0] - m) * coeff
    part = jax.lax.dot_general(
        e, x_ref[...], (((1,), (0,)), ((), ())),
        preferred_element_type=jnp.float32)

    @pl.when(i == 0)
    def _init():
        out_ref[...] = jnp.zeros_like(out_ref)

    out_ref[...] += part


def _make_xt_kernel(n, t, b, nb):
    def body(stats_ref, tailp_ref, w_ref, xt_ref, out_ref,
             xa, xb_, wa, wb, sb, tb, acc, sxa, sxb, swa, swb):
        pltpu.sync_copy(stats_ref, sb)
        pltpu.sync_copy(tailp_ref, tb)
        m = sb[0, 0]
        coeff = sb[1, 0]

        xbufs = (xa, xb_)
        wbufs = (wa, wb)
        sxs = (sxa, sxb)
        sws = (swa, swb)

        def start(j, p):
            pltpu.async_copy(xt_ref.at[:, pl.ds(j * b, b)], xbufs[p], sxs[p])
            pltpu.async_copy(w_ref.at[pl.ds(j * b, b)], wbufs[p], sws[p])

        def wait(p):
            pltpu.make_async_copy(
                xt_ref.at[:, pl.ds(0, b)], xbufs[p], sxs[p]).wait()
            pltpu.make_async_copy(
                w_ref.at[pl.ds(0, b)], wbufs[p], sws[p]).wait()

        def process(p):
            e = (jnp.exp(wbufs[p][...] - m) * coeff).reshape(1, b)
            acc[...] += xbufs[p][...] * e

        start(0, 0)
        if nb > 1:
            start(1, 1)
        acc[...] = jnp.zeros_like(acc)

        def pair(i, carry):
            j0 = 2 * i
            wait(0)
            process(0)

            @pl.when(j0 + 2 < nb)
            def _():
                start(j0 + 2, 0)

            wait(1)
            process(1)

            @pl.when(j0 + 3 < nb)
            def _():
                start(j0 + 3, 1)

            return carry

        jax.lax.fori_loop(0, nb // 2, pair, 0)
        if nb % 2 == 1:
            wait((nb - 1) % 2)
            process((nb - 1) % 2)
        out_ref[...] = jnp.sum(acc[...], axis=1).reshape(1, t) + tb[...]

    return pl.pallas_call(
        body,
        out_shape=jax.ShapeDtypeStruct((1, t), jnp.float32),
        in_specs=[
            pl.BlockSpec(memory_space=pltpu.HBM),
            pl.BlockSpec(memory_space=pltpu.HBM),
            pl.BlockSpec(memory_space=pltpu.HBM),
            pl.BlockSpec(memory_space=pltpu.HBM),
        ],
        out_specs=pl.BlockSpec((1, t), lambda: (0, 0)),
        scratch_shapes=[
            pltpu.VMEM((t, b), jnp.float32),
            pltpu.VMEM((t, b), jnp.float32),
            pltpu.VMEM((b,), jnp.float32),
            pltpu.VMEM((b,), jnp.float32),
            pltpu.VMEM((2, 16), jnp.float32),
            pltpu.VMEM((1, t), jnp.float32),
            pltpu.VMEM((t, b), jnp.float32),
            pltpu.SemaphoreType.DMA,
            pltpu.SemaphoreType.DMA,
            pltpu.SemaphoreType.DMA,
            pltpu.SemaphoreType.DMA,
        ],
    )


def _pick_block(n):
    for b in (8000, 10000, 5000, 4096, 4000, 2048, 2000, 1000):
        if n % b == 0:
            return b
    return n


def kernel(x, w, k_param):
    n, t = x.shape
    rows = 1000 if n % 1000 == 0 else 1
    w2d = w.reshape(n // rows, rows)
    k2d = k_param.reshape(1, 1)

    bc = _COLS
    nb = n // bc
    tail = n - nb * bc
    use_xt = (t % 8 == 0 and nb >= 2 and tail % 8 == 0 and tail > 0
              and bc % 1024 == 0)

    if use_xt:
        wt = w[n - tail:].reshape(1, tail)
        xtail = x[n - tail:]
        stats, tailp = pl.pallas_call(
            _stats_tail_kernel,
            out_shape=(
                jax.ShapeDtypeStruct((2, 16), jnp.float32),
                jax.ShapeDtypeStruct((1, t), jnp.float32),
            ),
            in_specs=[
                pl.BlockSpec((n // rows, rows), lambda: (0, 0)),
                pl.BlockSpec((1, 1), lambda: (0, 0)),
                pl.BlockSpec((1, tail), lambda: (0, 0)),
                pl.BlockSpec((tail, t), lambda: (0, 0)),
            ],
            out_specs=(
                pl.BlockSpec((2, 16), lambda: (0, 0)),
                pl.BlockSpec((1, t), lambda: (0, 0)),
            ),
        )(w2d, k2d, wt, xtail)
        out = _make_xt_kernel(n, t, bc, nb)(stats, tailp, w, x.T)
    else:
        stats = pl.pallas_call(
            _stats_kernel,
            out_shape=jax.ShapeDtypeStruct((2, 16), jnp.float32),
            in_specs=[
                pl.BlockSpec((n // rows, rows), lambda: (0, 0)),
                pl.BlockSpec((1, 1), lambda: (0, 0)),
            ],
            out_specs=pl.BlockSpec((2, 16), lambda: (0, 0)),
        )(w2d, k2d)
        b = _pick_block(n)
        out = pl.pallas_call(
            _wsum_tc_grid_kernel,
            grid=(n // b,),
            out_shape=jax.ShapeDtypeStruct((1, t), jnp.float32),
            in_specs=[
                pl.BlockSpec((2, 16), lambda i: (0, 0)),
                pl.BlockSpec((1, 1, b), lambda i: (i, 0, 0)),
                pl.BlockSpec((b, t), lambda i: (i, 0)),
            ],
            out_specs=pl.BlockSpec((1, t), lambda i: (0, 0)),
        )(stats, w.reshape(n // b, 1, b), x)

    return out.reshape(t)
